# baseline (speedup 1.0000x reference)
import jax
import jax.numpy as jnp
from jax.experimental import pallas as pl


def _copy_body(emb_ref, out_ref):
    out_ref[0] = emb_ref[...]


def kernel(position_ids, position_embeddings):
    B, S, H = position_ids.shape
    block_s = 512
    grid = (S // block_s, B)
    out = pl.pallas_call(
        _copy_body,
        grid=grid,
        in_specs=[pl.BlockSpec((block_s, H), lambda i, b: (i, 0))],
        out_specs=pl.BlockSpec((1, block_s, H), lambda i, b: (b, i, 0)),
        out_shape=jax.ShapeDtypeStruct((B, S, H), jnp.float32),
    )(position_embeddings[:S])
    return out


# speedup vs baseline: 1.4695x; 1.4695x over previous
import jax
import jax.numpy as jnp
from jax.experimental import pallas as pl


def _bcast_body(emb_ref, out_ref):
    out_ref[...] = jnp.broadcast_to(emb_ref[...][None], out_ref.shape)


def kernel(position_ids, position_embeddings):
    B, S, H = position_ids.shape
    block_s = 512
    grid = (S // block_s,)
    out = pl.pallas_call(
        _bcast_body,
        grid=grid,
        in_specs=[pl.BlockSpec((block_s, H), lambda i: (i, 0))],
        out_specs=pl.BlockSpec((B, block_s, H), lambda i: (0, i, 0)),
        out_shape=jax.ShapeDtypeStruct((B, S, H), jnp.float32),
    )(position_embeddings[:S])
    return out
